# baseline (speedup 1.0000x reference)
import jax
import jax.numpy as jnp
from jax.experimental import pallas as pl
from jax.experimental.pallas import tpu as pltpu

R = 128
C = 100000
W = 1024
NBLK = -(-C // W)

_ROT0 = (13, 15, 26, 6)
_ROT1 = (17, 29, 16, 24)


def _rotl(x, r):
    return (x << jnp.uint32(r)) | (x >> jnp.uint32(32 - r))


def _threefry_bits(n):
    ks0 = jnp.uint32(0)
    ks1 = jnp.uint32(42)
    ks2 = ks0 ^ ks1 ^ jnp.uint32(0x1BD11BDA)
    ks = (ks0, ks1, ks2)
    x0 = jnp.zeros_like(n) + ks0
    x1 = n + ks1
    for g in range(5):
        rots = _ROT0 if g % 2 == 0 else _ROT1
        for r in rots:
            x0 = x0 + x1
            x1 = _rotl(x1, r)
            x1 = x1 ^ x0
        x0 = x0 + ks[(g + 1) % 3]
        x1 = x1 + ks[(g + 2) % 3] + jnp.uint32(g + 1)
    return x0 ^ x1


def _partial_body(x_ref, val_ref, idx_ref):
    b = pl.program_id(0)

    x = x_ref[...]
    m = jnp.max(x, axis=0, keepdims=True)
    e = jnp.exp(x - m)
    s = jnp.sum(e, axis=0, keepdims=True)

    row = jax.lax.broadcasted_iota(jnp.uint32, (R, W), 0)
    col = jax.lax.broadcasted_iota(jnp.uint32, (R, W), 1)
    gcol = jnp.uint32(W) * b.astype(jnp.uint32) + col
    n = row * jnp.uint32(C) + gcol
    bits = _threefry_bits(n)

    tiny = jnp.float32(jnp.finfo(jnp.float32).tiny)
    fb = (bits >> jnp.uint32(9)) | jnp.uint32(0x3F800000)
    f = jax.lax.bitcast_convert_type(fb, jnp.float32) - jnp.float32(1.0)
    u = jnp.maximum(f, tiny)
    t = -jnp.log(u)

    score = e / (s * t)
    score = jnp.where(gcol < jnp.uint32(C), score, jnp.float32(-1.0))

    bm = jnp.max(score, axis=1, keepdims=True)
    is_max = score == bm
    cand_idx = jnp.min(
        jnp.where(is_max, gcol.astype(jnp.int32), jnp.int32(0x7FFFFFFF)),
        axis=1, keepdims=True)

    val_ref[...] = jnp.reshape(bm, (1, 1, R))
    idx_ref[...] = jnp.reshape(cand_idx, (1, 1, R))


def _merge_body(val_ref, idx_ref, o_ref):
    v = val_ref[...]
    i = idx_ref[...]
    bm = jnp.max(v, axis=0)
    o_ref[...] = jnp.min(
        jnp.where(v == bm[None], i, jnp.int32(0x7FFFFFFF)), axis=0)


@jax.jit
def kernel(outputs):
    val, idx = pl.pallas_call(
        _partial_body,
        grid=(NBLK,),
        in_specs=[pl.BlockSpec((R, W), lambda b: (0, b))],
        out_specs=[
            pl.BlockSpec((1, 1, R), lambda b: (b, 0, 0)),
            pl.BlockSpec((1, 1, R), lambda b: (b, 0, 0)),
        ],
        out_shape=[
            jax.ShapeDtypeStruct((NBLK, 1, R), jnp.float32),
            jax.ShapeDtypeStruct((NBLK, 1, R), jnp.int32),
        ],
        compiler_params=pltpu.CompilerParams(
            dimension_semantics=("parallel",),
        ),
    )(outputs)
    out = pl.pallas_call(
        _merge_body,
        out_shape=jax.ShapeDtypeStruct((1, R), jnp.int32),
    )(val, idx)
    return jnp.reshape(out, (R, 1))


# speedup vs baseline: 1.0157x; 1.0157x over previous
import jax
import jax.numpy as jnp
from jax.experimental import pallas as pl
from jax.experimental.pallas import tpu as pltpu

R = 128
C = 100000
W = 1024
NBLK = -(-C // W)

_ROT0 = (13, 15, 26, 6)
_ROT1 = (17, 29, 16, 24)


def _rotl(x, r):
    return (x << jnp.uint32(r)) | (x >> jnp.uint32(32 - r))


def _threefry_bits(x1):
    ks0 = jnp.uint32(0)
    ks1 = jnp.uint32(42)
    ks2 = ks0 ^ ks1 ^ jnp.uint32(0x1BD11BDA)
    ks = (ks0, ks1, ks2)
    x0 = jnp.zeros_like(x1)
    for g in range(5):
        rots = _ROT0 if g % 2 == 0 else _ROT1
        for r in rots:
            x0 = x0 + x1
            x1 = _rotl(x1, r)
            x1 = x1 ^ x0
        x0 = x0 + ks[(g + 1) % 3]
        x1 = x1 + ks[(g + 2) % 3] + jnp.uint32(g + 1)
    return x0 ^ x1


def _body(x_ref, o_ref, val_ref, idx_ref, x1c_ref):
    b = pl.program_id(0)

    @pl.when(b == 0)
    def _init():
        val_ref[...] = jnp.full((R, 1), -1.0, jnp.float32)
        idx_ref[...] = jnp.zeros((R, 1), jnp.int32)
        row = jax.lax.broadcasted_iota(jnp.uint32, (R, W), 0)
        col0 = jax.lax.broadcasted_iota(jnp.uint32, (R, W), 1)
        x1c_ref[...] = row * jnp.uint32(C) + col0 + jnp.uint32(42)

    x = x_ref[...]
    m = jnp.max(x, axis=0, keepdims=True)
    e = jnp.exp(x - m)
    s = jax.lax.dot_general(
        jnp.ones((1, R), jnp.float32), e,
        dimension_numbers=(((1,), (0,)), ((), ())),
        preferred_element_type=jnp.float32)

    x1 = x1c_ref[...]
    x1c_ref[...] = x1 + jnp.uint32(W)
    bits = _threefry_bits(x1)

    tiny = jnp.float32(jnp.finfo(jnp.float32).tiny)
    fb = (bits >> jnp.uint32(9)) | jnp.uint32(0x3F800000)
    f = jax.lax.bitcast_convert_type(fb, jnp.float32) - jnp.float32(1.0)
    u = jnp.maximum(f, tiny)
    t = -jnp.log(u)

    score = e / (s * t)
    col = jax.lax.broadcasted_iota(jnp.int32, (1, W), 1)
    valid = col < (jnp.int32(C) - jnp.int32(W) * b)
    score = jnp.where(valid, score, jnp.float32(-1.0))

    bm = jnp.max(score, axis=1, keepdims=True)
    is_max = score == bm
    lidx = jnp.min(
        jnp.where(is_max, jax.lax.broadcasted_iota(jnp.int32, (R, W), 1),
                  jnp.int32(0x7FFFFFFF)),
        axis=1, keepdims=True)
    cand_idx = lidx + jnp.int32(W) * b

    better = bm > val_ref[...]
    val_ref[...] = jnp.where(better, bm, val_ref[...])
    idx_ref[...] = jnp.where(better, cand_idx, idx_ref[...])

    @pl.when(b == NBLK - 1)
    def _emit():
        o_ref[...] = idx_ref[...]


@jax.jit
def kernel(outputs):
    return pl.pallas_call(
        _body,
        grid=(NBLK,),
        in_specs=[pl.BlockSpec((R, W), lambda b: (0, b))],
        out_specs=pl.BlockSpec((R, 1), lambda b: (0, 0)),
        out_shape=jax.ShapeDtypeStruct((R, 1), jnp.int32),
        scratch_shapes=[
            pltpu.VMEM((R, 1), jnp.float32),
            pltpu.VMEM((R, 1), jnp.int32),
            pltpu.VMEM((R, W), jnp.uint32),
        ],
    )(outputs)
